# Initial kernel scaffold; baseline (speedup 1.0000x reference)
import jax
import jax.numpy as jnp
from jax.experimental import pallas as pl


def kernel(x, x1, params, edge, edge1, batch):
    raise NotImplementedError("write your pallas kernel here")


# speedup vs baseline: 16.1580x; 16.1580x over previous
import functools

import jax
import jax.numpy as jnp
from jax import lax
from jax.experimental import pallas as pl
from jax.experimental.pallas import tpu as pltpu
from jax.experimental.pallas import tpu_sc as plsc

N = 10000
E = 320000
D = 128
G = 64
TAU = 0.2

NC = 2
NS = 16
EPT = E // NS
CH = 128
NCH = 160
EPP = NCH * CH
NP = 10240
RPT = NP // NS

_SC_MESH = dict(core_axis_name="c", subcore_axis_name="s")


QC = NCH // 4


def _sc_scatter_body(hp, srcs, dsts, out, acc, sidx, didx, r0, r1,
                     sem0, sem1):
    c = lax.axis_index("c")
    s = lax.axis_index("s")

    def _zrow(r, _):
        for j in range(D // 16):
            r0[r, pl.ds(j * 16, 16)] = jnp.zeros((16,), jnp.float32)
        return 0

    lax.fori_loop(0, CH, _zrow, 0)
    for k in range(RPT // CH):
        pltpu.sync_copy(r0, acc.at[pl.ds(s * RPT + k * CH, CH)])

    w = (c * NS + s) * NCH
    plsc.subcore_barrier()

    for q in range(NCH // QC):
        pltpu.sync_copy(srcs.at[pl.ds(w + q * QC, QC)], sidx)
        pltpu.sync_copy(dsts.at[pl.ds(w + q * QC, QC)], didx)
        pltpu.async_copy(hp.at[sidx.at[0]], r0, sem0)

        def _pair(p, _):
            i0 = 2 * p
            pltpu.async_copy(hp.at[sidx.at[i0 + 1]], r1, sem1)
            pltpu.make_async_copy(hp.at[sidx.at[i0]], r0, sem0).wait()
            pltpu.sync_copy(r0, acc.at[didx.at[i0]], add=True)
            pltpu.async_copy(hp.at[sidx.at[i0 + 2]], r0, sem0)
            pltpu.make_async_copy(hp.at[sidx.at[i0 + 1]], r1, sem1).wait()
            pltpu.sync_copy(r1, acc.at[didx.at[i0 + 1]], add=True)
            return 0

        lax.fori_loop(0, QC // 2 - 1, _pair, 0)
        pltpu.async_copy(hp.at[sidx.at[QC - 1]], r1, sem1)
        pltpu.make_async_copy(hp.at[sidx.at[QC - 2]], r0, sem0).wait()
        pltpu.sync_copy(r0, acc.at[didx.at[QC - 2]], add=True)
        pltpu.make_async_copy(hp.at[sidx.at[QC - 1]], r1, sem1).wait()
        pltpu.sync_copy(r1, acc.at[didx.at[QC - 1]], add=True)

    plsc.subcore_barrier()
    pltpu.sync_copy(acc.at[pl.ds(s * RPT, RPT)],
                    out.at[pl.ds(c * NP + s * RPT, RPT)])


@functools.cache
def _sc_scatter_kernel():
    return pl.kernel(
        _sc_scatter_body,
        out_type=jax.ShapeDtypeStruct((2 * NP, D), jnp.float32),
        mesh=plsc.VectorSubcoreMesh(**_SC_MESH),
        scratch_types=[
            pltpu.VMEM_SHARED((NP, D), jnp.float32),
            pltpu.VMEM((QC, CH), jnp.int32),
            pltpu.VMEM((QC, CH), jnp.int32),
            pltpu.VMEM((CH, D), jnp.float32),
            pltpu.VMEM((CH, D), jnp.float32),
            pltpu.SemaphoreType.DMA,
            pltpu.SemaphoreType.DMA,
        ],
    )


def _sc_scatter(hp, srcs, dsts):
    return _sc_scatter_kernel()(hp, srcs, dsts)


_RB = 400


def _dotT(a, b):
    return lax.dot_general(a, b, (((1,), (1,)), ((), ())),
                           preferred_element_type=jnp.float32)


def _mm_scale_body(x_ref, w_ref, cnt_ref, o_ref):
    dinv = lax.rsqrt(1.0 + cnt_ref[:, 0:1])
    o_ref[...] = _dotT(x_ref[...], w_ref[...]) * dinv


def _mm_scale(xs, w, cnt):
    grid = (2 * N) // _RB
    return pl.pallas_call(
        _mm_scale_body,
        grid=(grid,),
        in_specs=[
            pl.BlockSpec((_RB, D), lambda i: (i, 0)),
            pl.BlockSpec((D, D), lambda i: (0, 0)),
            pl.BlockSpec((_RB, D), lambda i: (i, 0)),
        ],
        out_specs=pl.BlockSpec((_RB, D), lambda i: (i, 0)),
        out_shape=jax.ShapeDtypeStruct((2 * N, D), jnp.float32),
    )(xs, w, cnt)


def _combine_body(msg_ref, hp_ref, cnt_ref, b_ref, a_ref, o_ref):
    dinv = lax.rsqrt(1.0 + cnt_ref[:, 0:1])
    t = dinv * (msg_ref[...] + hp_ref[...]) + b_ref[...]
    o_ref[...] = jnp.where(t >= 0.0, t, a_ref[...] * t)


def _combine_prelu(msg, hp, cnt, b, a):
    grid = (2 * N) // _RB
    return pl.pallas_call(
        _combine_body,
        grid=(grid,),
        in_specs=[
            pl.BlockSpec((_RB, D), lambda i: (i, 0)),
            pl.BlockSpec((_RB, D), lambda i: (i, 0)),
            pl.BlockSpec((_RB, D), lambda i: (i, 0)),
            pl.BlockSpec((1, D), lambda i: (0, 0)),
            pl.BlockSpec((1, D), lambda i: (0, 0)),
        ],
        out_specs=pl.BlockSpec((_RB, D), lambda i: (i, 0)),
        out_shape=jax.ShapeDtypeStruct((2 * N, D), jnp.float32),
    )(msg, hp, cnt, b, a)


_SB = 512
_NP = 10240


def _segmax_body(z_ref, b_ref, o_ref):
    j = pl.program_id(1)

    @pl.when(j == 0)
    def _():
        o_ref[...] = jnp.full((1, G, D), -jnp.inf, jnp.float32)

    zb = z_ref[0]
    bb = b_ref[0]
    gmin = jnp.min(bb)
    gmax = jnp.minimum(jnp.max(bb), G - 1)

    def _upd(g, acc):
        red = jnp.max(jnp.where(bb == g, zb, -jnp.inf), axis=0, keepdims=True)
        sel = lax.broadcasted_iota(jnp.int32, (G, 1), 0) == g
        return jnp.where(sel, jnp.maximum(acc, red), acc)

    o_ref[0] = lax.fori_loop(gmin, gmax + 1, _upd, o_ref[0])


def _segmax(zp, batch3):
    return pl.pallas_call(
        _segmax_body,
        grid=(2, _NP // _SB),
        in_specs=[
            pl.BlockSpec((1, _SB, D), lambda p, j: (p, j, 0)),
            pl.BlockSpec((1, _SB, 1), lambda p, j: (j, 0, 0)),
        ],
        out_specs=pl.BlockSpec((1, G, D), lambda p, j: (p, 0, 0)),
        out_shape=jax.ShapeDtypeStruct((2, G, D), jnp.float32),
    )(zp, batch3)


def _fc_body(x_ref, w1, b1, w2, b2, w3, b3, ws, bs, o_ref):
    x = x_ref[...]
    h = jnp.maximum(_dotT(x, w1[...]) + b1[...], 0.0)
    h = jnp.maximum(_dotT(h, w2[...]) + b2[...], 0.0)
    h = jnp.maximum(_dotT(h, w3[...]) + b3[...], 0.0)
    o_ref[...] = h + _dotT(x, ws[...]) + bs[...]


def _fc_block(x, p, rb):
    m = x.shape[0]
    args = [x]
    specs = [pl.BlockSpec((rb, D), lambda i: (i, 0))]
    for k in ("w1", "b1", "w2", "b2", "w3", "b3", "ws", "bs"):
        v = p[k]
        if v.ndim == 1:
            v = v.reshape(1, D)
            specs.append(pl.BlockSpec((1, D), lambda i: (0, 0)))
        else:
            specs.append(pl.BlockSpec((D, D), lambda i: (0, 0)))
        args.append(v)
    return pl.pallas_call(
        _fc_body,
        grid=(m // rb,),
        in_specs=specs,
        out_specs=pl.BlockSpec((rb, D), lambda i: (i, 0)),
        out_shape=jax.ShapeDtypeStruct((m, D), jnp.float32),
    )(*args)


def _infonce_body(an_ref, sm_ref, b_ref, o_ref, accA, accB, accC):
    j = pl.program_id(1)

    @pl.when(j == 0)
    def _():
        accA[...] = jnp.zeros_like(accA)
        accB[...] = jnp.zeros_like(accB)
        accC[...] = jnp.zeros_like(accC)

    an = an_ref[0]
    an = an / (jnp.sqrt(jnp.sum(an * an, axis=1, keepdims=True)) + 1e-12)
    sm = sm_ref[0]
    sm = sm / (jnp.sqrt(jnp.sum(sm * sm, axis=1, keepdims=True)) + 1e-12)
    sim = _dotT(an, sm) / TAU
    mask = b_ref[0] == lax.broadcasted_iota(jnp.int32, (G, 1), 0)
    accA[...] += jnp.sum(jnp.exp(sim), axis=1, keepdims=True)
    accB[...] += jnp.sum(jnp.where(mask, sim, 0.0), axis=1, keepdims=True)
    accC[...] += jnp.sum(mask.astype(jnp.float32), axis=1, keepdims=True)

    @pl.when(j == pl.num_programs(1) - 1)
    def _():
        per = accB[...] / accC[...] - jnp.log(accA[...])
        o_ref[...] = jnp.broadcast_to(-jnp.sum(per) / G, (1, 8, D))


def _infonce(anchors, samples, batch3i):
    return pl.pallas_call(
        _infonce_body,
        grid=(2, N // _RB),
        in_specs=[
            pl.BlockSpec((1, G, D), lambda p, j: (p, 0, 0)),
            pl.BlockSpec((1, _RB, D), lambda p, j: (p, j, 0)),
            pl.BlockSpec((1, 1, _RB), lambda p, j: (j, 0, 0)),
        ],
        out_specs=pl.BlockSpec((1, 8, D), lambda p, j: (p, 0, 0)),
        out_shape=jax.ShapeDtypeStruct((2, 8, D), jnp.float32),
        scratch_shapes=[
            pltpu.VMEM((G, 1), jnp.float32),
            pltpu.VMEM((G, 1), jnp.float32),
            pltpu.VMEM((G, 1), jnp.float32),
        ],
    )(anchors, samples, batch3i)


def _unpad(v):
    return jnp.concatenate([v[:N], v[NP:NP + N]])


def _chunked(idx, pad):
    body = idx.reshape(NS, EPT)
    tail = jnp.broadcast_to(pad, (NS, EPP - EPT))
    return jnp.concatenate([body, tail], axis=1).reshape(NS * NCH, CH)


def kernel(x, x1, params, edge, edge1, batch):
    dpad = N + jnp.arange(EPP - EPT, dtype=jnp.int32) % (NP - N)
    spad = jnp.arange(EPP - EPT, dtype=jnp.int32) % N
    srcs = jnp.concatenate(
        [_chunked(edge[0], spad), _chunked(edge1[0] + N, spad + N)])
    dsts = jnp.concatenate(
        [_chunked(edge[1], dpad), _chunked(edge1[1], dpad)])

    cnt = _unpad(_sc_scatter(jnp.ones((2 * N, D), jnp.float32), srcs, dsts))

    xs = jnp.concatenate([x, x1], axis=0)
    a = params["prelu_a"].reshape(1, D)
    for i in range(2):
        w = params["conv%d_w" % i]
        b = params["conv%d_b" % i].reshape(1, D)
        hp = _mm_scale(xs, w, cnt)
        msg = _unpad(_sc_scatter(hp, srcs, dsts))
        xs = _combine_prelu(msg, hp, cnt, b, a)

    zp = jnp.concatenate(
        [xs.reshape(2, N, D),
         jnp.zeros((2, _NP - N, D), jnp.float32)], axis=1)
    batch_pad = jnp.concatenate(
        [batch, jnp.full((_NP - N,), G, jnp.int32)]).reshape(_NP // _SB, _SB, 1)
    gs = _segmax(zp, batch_pad)

    pro_zs = _fc_block(xs, params["local"], _RB)
    pro_gs = _fc_block(gs.reshape(2 * G, D), params["global"], 2 * G)

    anchors = pro_gs.reshape(2, G, D)
    samples = jnp.stack([pro_zs[N:], pro_zs[:N]])
    batch3i = batch.reshape(N // _RB, 1, _RB)
    ls = _infonce(anchors, samples, batch3i)
    loss = 0.5 * (ls[0, 0, 0] + ls[1, 0, 0])
    return (loss, xs[:N], gs[0])


# speedup vs baseline: 17.9060x; 1.1082x over previous
import functools

import jax
import jax.numpy as jnp
from jax import lax
from jax.experimental import pallas as pl
from jax.experimental.pallas import tpu as pltpu
from jax.experimental.pallas import tpu_sc as plsc

N = 10000
E = 320000
D = 128
G = 64
TAU = 0.2

NC = 2
NS = 16
EPT = E // NS
CH = 128
NCH = 160
EPP = NCH * CH
NP = 10240
RPT = NP // NS

_SC_MESH = dict(core_axis_name="c", subcore_axis_name="s")


QC = NCH // 4


def _sc_scatter_body(hp, srcs, dsts, out, acc, sidx, didx, r0, r1,
                     sem0, sem1):
    c = lax.axis_index("c")
    s = lax.axis_index("s")

    def _zrow(r, _):
        for j in range(D // 16):
            r0[r, pl.ds(j * 16, 16)] = jnp.zeros((16,), jnp.float32)
        return 0

    lax.fori_loop(0, CH, _zrow, 0)
    for k in range(RPT // CH):
        pltpu.sync_copy(r0, acc.at[pl.ds(s * RPT + k * CH, CH)])

    w = (c * NS + s) * NCH
    plsc.subcore_barrier()

    for q in range(NCH // QC):
        pltpu.sync_copy(srcs.at[pl.ds(w + q * QC, QC)], sidx)
        pltpu.sync_copy(dsts.at[pl.ds(w + q * QC, QC)], didx)
        pltpu.async_copy(hp.at[sidx.at[0]], r0, sem0)

        def _pair(p, _):
            i0 = 2 * p
            pltpu.async_copy(hp.at[sidx.at[i0 + 1]], r1, sem1)
            pltpu.make_async_copy(hp.at[sidx.at[i0]], r0, sem0).wait()
            pltpu.sync_copy(r0, acc.at[didx.at[i0]], add=True)
            pltpu.async_copy(hp.at[sidx.at[i0 + 2]], r0, sem0)
            pltpu.make_async_copy(hp.at[sidx.at[i0 + 1]], r1, sem1).wait()
            pltpu.sync_copy(r1, acc.at[didx.at[i0 + 1]], add=True)
            return 0

        lax.fori_loop(0, QC // 2 - 1, _pair, 0)
        pltpu.async_copy(hp.at[sidx.at[QC - 1]], r1, sem1)
        pltpu.make_async_copy(hp.at[sidx.at[QC - 2]], r0, sem0).wait()
        pltpu.sync_copy(r0, acc.at[didx.at[QC - 2]], add=True)
        pltpu.make_async_copy(hp.at[sidx.at[QC - 1]], r1, sem1).wait()
        pltpu.sync_copy(r1, acc.at[didx.at[QC - 1]], add=True)

    plsc.subcore_barrier()
    pltpu.sync_copy(acc.at[pl.ds(s * RPT, RPT)],
                    out.at[pl.ds(c * NP + s * RPT, RPT)])


@functools.cache
def _sc_scatter_kernel():
    return pl.kernel(
        _sc_scatter_body,
        out_type=jax.ShapeDtypeStruct((2 * NP, D), jnp.float32),
        mesh=plsc.VectorSubcoreMesh(**_SC_MESH),
        scratch_types=[
            pltpu.VMEM_SHARED((NP, D), jnp.float32),
            pltpu.VMEM((QC, CH), jnp.int32),
            pltpu.VMEM((QC, CH), jnp.int32),
            pltpu.VMEM((CH, D), jnp.float32),
            pltpu.VMEM((CH, D), jnp.float32),
            pltpu.SemaphoreType.DMA,
            pltpu.SemaphoreType.DMA,
        ],
    )


def _sc_scatter(hp, srcs, dsts):
    return _sc_scatter_kernel()(hp, srcs, dsts)


_RB = 400


def _dotT(a, b):
    return lax.dot_general(a, b, (((1,), (1,)), ((), ())),
                           preferred_element_type=jnp.float32)


def _mm_scale_body(x_ref, w_ref, cnt_ref, o_ref):
    dinv = lax.rsqrt(1.0 + cnt_ref[:, 0:1])
    o_ref[...] = _dotT(x_ref[...], w_ref[...]) * dinv


def _mm_scale(xs, w, cnt):
    grid = (2 * N) // _RB
    return pl.pallas_call(
        _mm_scale_body,
        grid=(grid,),
        in_specs=[
            pl.BlockSpec((_RB, D), lambda i: (i, 0)),
            pl.BlockSpec((D, D), lambda i: (0, 0)),
            pl.BlockSpec((_RB, D), lambda i: (i, 0)),
        ],
        out_specs=pl.BlockSpec((_RB, D), lambda i: (i, 0)),
        out_shape=jax.ShapeDtypeStruct((2 * N, D), jnp.float32),
    )(xs, w, cnt)


def _combine_mm_body(msg_ref, hp_ref, cnt_ref, b_ref, a_ref, w_ref, o_ref):
    dinv = lax.rsqrt(1.0 + cnt_ref[:, 0:1])
    t = dinv * (msg_ref[...] + hp_ref[...]) + b_ref[...]
    z = jnp.where(t >= 0.0, t, a_ref[...] * t)
    o_ref[...] = _dotT(z, w_ref[...]) * dinv


def _combine_mm(msg, hp, cnt, b, a, w):
    grid = (2 * N) // _RB
    return pl.pallas_call(
        _combine_mm_body,
        grid=(grid,),
        in_specs=[
            pl.BlockSpec((_RB, D), lambda i: (i, 0)),
            pl.BlockSpec((_RB, D), lambda i: (i, 0)),
            pl.BlockSpec((_RB, D), lambda i: (i, 0)),
            pl.BlockSpec((1, D), lambda i: (0, 0)),
            pl.BlockSpec((1, D), lambda i: (0, 0)),
            pl.BlockSpec((D, D), lambda i: (0, 0)),
        ],
        out_specs=pl.BlockSpec((_RB, D), lambda i: (i, 0)),
        out_shape=jax.ShapeDtypeStruct((2 * N, D), jnp.float32),
    )(msg, hp, cnt, b, a, w)


_GB = N // _RB


def _combine_segmax_body(msg_ref, hp_ref, cnt_ref, b_ref, a_ref, bb_ref,
                         z_ref, g_ref):
    i = pl.program_id(0)
    dinv = lax.rsqrt(1.0 + cnt_ref[:, 0:1])
    t = dinv * (msg_ref[...] + hp_ref[...]) + b_ref[...]
    z = jnp.where(t >= 0.0, t, a_ref[...] * t)
    z_ref[...] = z

    @pl.when(i % _GB == 0)
    def _():
        g_ref[...] = jnp.full((1, G, D), -jnp.inf, jnp.float32)

    bb = bb_ref[0]
    gmin = jnp.min(bb)
    gmax = jnp.max(bb)

    def _upd(g, acc):
        red = jnp.max(jnp.where(bb == g, z, -jnp.inf), axis=0, keepdims=True)
        sel = lax.broadcasted_iota(jnp.int32, (G, 1), 0) == g
        return jnp.where(sel, jnp.maximum(acc, red), acc)

    g_ref[0] = lax.fori_loop(gmin, gmax + 1, _upd, g_ref[0])


def _combine_segmax(msg, hp, cnt, b, a, batchc):
    return pl.pallas_call(
        _combine_segmax_body,
        grid=(2 * _GB,),
        in_specs=[
            pl.BlockSpec((_RB, D), lambda i: (i, 0)),
            pl.BlockSpec((_RB, D), lambda i: (i, 0)),
            pl.BlockSpec((_RB, D), lambda i: (i, 0)),
            pl.BlockSpec((1, D), lambda i: (0, 0)),
            pl.BlockSpec((1, D), lambda i: (0, 0)),
            pl.BlockSpec((1, _RB, 1), lambda i: (i % _GB, 0, 0)),
        ],
        out_specs=[
            pl.BlockSpec((_RB, D), lambda i: (i, 0)),
            pl.BlockSpec((1, G, D), lambda i: (i // _GB, 0, 0)),
        ],
        out_shape=[
            jax.ShapeDtypeStruct((2 * N, D), jnp.float32),
            jax.ShapeDtypeStruct((2, G, D), jnp.float32),
        ],
    )(msg, hp, cnt, b, a, batchc)


def _fc_body(x_ref, w1, b1, w2, b2, w3, b3, ws, bs, o_ref):
    x = x_ref[...]
    h = jnp.maximum(_dotT(x, w1[...]) + b1[...], 0.0)
    h = jnp.maximum(_dotT(h, w2[...]) + b2[...], 0.0)
    h = jnp.maximum(_dotT(h, w3[...]) + b3[...], 0.0)
    o_ref[...] = h + _dotT(x, ws[...]) + bs[...]


def _fc_block(x, p, rb):
    m = x.shape[0]
    args = [x]
    specs = [pl.BlockSpec((rb, D), lambda i: (i, 0))]
    for k in ("w1", "b1", "w2", "b2", "w3", "b3", "ws", "bs"):
        v = p[k]
        if v.ndim == 1:
            v = v.reshape(1, D)
            specs.append(pl.BlockSpec((1, D), lambda i: (0, 0)))
        else:
            specs.append(pl.BlockSpec((D, D), lambda i: (0, 0)))
        args.append(v)
    return pl.pallas_call(
        _fc_body,
        grid=(m // rb,),
        in_specs=specs,
        out_specs=pl.BlockSpec((rb, D), lambda i: (i, 0)),
        out_shape=jax.ShapeDtypeStruct((m, D), jnp.float32),
    )(*args)


def _infonce_body(an_ref, z_ref, b_ref, w1, b1, w2, b2, w3, b3, ws, bs,
                  o_ref, accA, accB, accC):
    j = pl.program_id(1)

    @pl.when(j == 0)
    def _():
        accA[...] = jnp.zeros_like(accA)
        accB[...] = jnp.zeros_like(accB)
        accC[...] = jnp.zeros_like(accC)

    an = an_ref[0]
    an = an / (jnp.sqrt(jnp.sum(an * an, axis=1, keepdims=True)) + 1e-12)
    x = z_ref[...]
    h = jnp.maximum(_dotT(x, w1[...]) + b1[...], 0.0)
    h = jnp.maximum(_dotT(h, w2[...]) + b2[...], 0.0)
    h = jnp.maximum(_dotT(h, w3[...]) + b3[...], 0.0)
    sm = h + _dotT(x, ws[...]) + bs[...]
    sm = sm / (jnp.sqrt(jnp.sum(sm * sm, axis=1, keepdims=True)) + 1e-12)
    sim = _dotT(an, sm) / TAU
    mask = b_ref[0] == lax.broadcasted_iota(jnp.int32, (G, 1), 0)
    accA[...] += jnp.sum(jnp.exp(sim), axis=1, keepdims=True)
    accB[...] += jnp.sum(jnp.where(mask, sim, 0.0), axis=1, keepdims=True)
    accC[...] += jnp.sum(mask.astype(jnp.float32), axis=1, keepdims=True)

    @pl.when(j == pl.num_programs(1) - 1)
    def _():
        per = accB[...] / accC[...] - jnp.log(accA[...])
        o_ref[...] = jnp.broadcast_to(-jnp.sum(per) / G, (1, 8, D))


def _infonce(anchors, zs, batch3i, prm):
    args = [anchors, zs, batch3i]
    specs = [
        pl.BlockSpec((1, G, D), lambda p, j: (p, 0, 0)),
        pl.BlockSpec((_RB, D), lambda p, j: ((1 - p) * _GB + j, 0)),
        pl.BlockSpec((1, 1, _RB), lambda p, j: (j, 0, 0)),
    ]
    for k in ("w1", "b1", "w2", "b2", "w3", "b3", "ws", "bs"):
        v = prm[k]
        if v.ndim == 1:
            v = v.reshape(1, D)
            specs.append(pl.BlockSpec((1, D), lambda p, j: (0, 0)))
        else:
            specs.append(pl.BlockSpec((D, D), lambda p, j: (0, 0)))
        args.append(v)
    return pl.pallas_call(
        _infonce_body,
        grid=(2, N // _RB),
        in_specs=specs,
        out_specs=pl.BlockSpec((1, 8, D), lambda p, j: (p, 0, 0)),
        out_shape=jax.ShapeDtypeStruct((2, 8, D), jnp.float32),
        scratch_shapes=[
            pltpu.VMEM((G, 1), jnp.float32),
            pltpu.VMEM((G, 1), jnp.float32),
            pltpu.VMEM((G, 1), jnp.float32),
        ],
    )(*args)


def _unpad(v):
    return jnp.concatenate([v[:N], v[NP:NP + N]])


def _chunked(idx, pad):
    body = idx.reshape(NS, EPT)
    tail = jnp.broadcast_to(pad, (NS, EPP - EPT))
    return jnp.concatenate([body, tail], axis=1).reshape(NS * NCH, CH)


def kernel(x, x1, params, edge, edge1, batch):
    dpad = N + jnp.arange(EPP - EPT, dtype=jnp.int32) % (NP - N)
    spad = jnp.arange(EPP - EPT, dtype=jnp.int32) % N
    srcs = jnp.concatenate(
        [_chunked(edge[0], spad), _chunked(edge1[0] + N, spad + N)])
    dsts = jnp.concatenate(
        [_chunked(edge[1], dpad), _chunked(edge1[1], dpad)])

    cnt = _unpad(_sc_scatter(jnp.ones((2 * N, D), jnp.float32), srcs, dsts))

    xs = jnp.concatenate([x, x1], axis=0)
    a = params["prelu_a"].reshape(1, D)
    b0 = params["conv0_b"].reshape(1, D)
    b1 = params["conv1_b"].reshape(1, D)

    hp = _mm_scale(xs, params["conv0_w"], cnt)
    msg = _unpad(_sc_scatter(hp, srcs, dsts))
    hp = _combine_mm(msg, hp, cnt, b0, a, params["conv1_w"])
    msg = _unpad(_sc_scatter(hp, srcs, dsts))
    zs, gs = _combine_segmax(msg, hp, cnt, b1, a,
                             batch.reshape(_GB, _RB, 1))

    pro_gs = _fc_block(gs.reshape(2 * G, D), params["global"], 2 * G)

    batch3i = batch.reshape(N // _RB, 1, _RB)
    ls = _infonce(pro_gs.reshape(2, G, D), zs, batch3i, params["local"])
    loss = 0.5 * (ls[0, 0, 0] + ls[1, 0, 0])
    return (loss, zs[:N], gs[0])


# speedup vs baseline: 20.0225x; 1.1182x over previous
import functools

import jax
import jax.numpy as jnp
from jax import lax
from jax.experimental import pallas as pl
from jax.experimental.pallas import tpu as pltpu
from jax.experimental.pallas import tpu_sc as plsc

N = 10000
E = 320000
D = 128
G = 64
TAU = 0.2

NC = 2
NS = 16
EPT = E // NS
CH = 128
NCH = 160
EPP = NCH * CH
NP = 10240
RPT = NP // NS

_SC_MESH = dict(core_axis_name="c", subcore_axis_name="s")


QC = NCH // 4


def _sc_scatter_body(hp, srcs, dsts, out, acc, sidx, didx, r0, r1,
                     sem0, sem1):
    c = lax.axis_index("c")
    s = lax.axis_index("s")

    def _zrow(r, _):
        for j in range(D // 16):
            r0[r, pl.ds(j * 16, 16)] = jnp.zeros((16,), jnp.float32)
        return 0

    lax.fori_loop(0, CH, _zrow, 0)
    for k in range(RPT // CH):
        pltpu.sync_copy(r0, acc.at[pl.ds(s * RPT + k * CH, CH)])

    w = (c * NS + s) * NCH
    plsc.subcore_barrier()

    for q in range(NCH // QC):
        pltpu.sync_copy(srcs.at[pl.ds(w + q * QC, QC)], sidx)
        pltpu.sync_copy(dsts.at[pl.ds(w + q * QC, QC)], didx)
        pltpu.async_copy(hp.at[sidx.at[0]], r0, sem0)

        def _pair(p, _):
            i0 = 2 * p
            pltpu.async_copy(hp.at[sidx.at[i0 + 1]], r1, sem1)
            pltpu.make_async_copy(hp.at[sidx.at[i0]], r0, sem0).wait()
            pltpu.sync_copy(r0, acc.at[didx.at[i0]], add=True)
            pltpu.async_copy(hp.at[sidx.at[i0 + 2]], r0, sem0)
            pltpu.make_async_copy(hp.at[sidx.at[i0 + 1]], r1, sem1).wait()
            pltpu.sync_copy(r1, acc.at[didx.at[i0 + 1]], add=True)
            return 0

        lax.fori_loop(0, QC // 2 - 1, _pair, 0)
        pltpu.async_copy(hp.at[sidx.at[QC - 1]], r1, sem1)
        pltpu.make_async_copy(hp.at[sidx.at[QC - 2]], r0, sem0).wait()
        pltpu.sync_copy(r0, acc.at[didx.at[QC - 2]], add=True)
        pltpu.make_async_copy(hp.at[sidx.at[QC - 1]], r1, sem1).wait()
        pltpu.sync_copy(r1, acc.at[didx.at[QC - 1]], add=True)

    plsc.subcore_barrier()
    pltpu.sync_copy(acc.at[pl.ds(s * RPT, RPT)],
                    out.at[pl.ds(c * NP + s * RPT, RPT)])


def _sc_count_body(dsts, out, acc, didx, r0):
    c = lax.axis_index("c")
    s = lax.axis_index("s")

    def _fill(val):
        def _row(r, _):
            for j in range(D // 16):
                r0[r, pl.ds(j * 16, 16)] = jnp.full((16,), val, jnp.float32)
            return 0
        lax.fori_loop(0, CH, _row, 0)

    _fill(0.0)
    for k in range(RPT // CH):
        pltpu.sync_copy(r0, acc.at[pl.ds(s * RPT + k * CH, CH)])
    _fill(1.0)

    w = (c * NS + s) * NCH
    plsc.subcore_barrier()
    for q in range(NCH // QC):
        pltpu.sync_copy(dsts.at[pl.ds(w + q * QC, QC)], didx)

        def _chunk(i, _):
            pltpu.sync_copy(r0, acc.at[didx.at[i]], add=True)
            return 0

        lax.fori_loop(0, QC, _chunk, 0)

    plsc.subcore_barrier()
    pltpu.sync_copy(acc.at[pl.ds(s * RPT, RPT)],
                    out.at[pl.ds(c * NP + s * RPT, RPT)])


@functools.cache
def _sc_count_kernel():
    return pl.kernel(
        _sc_count_body,
        out_type=jax.ShapeDtypeStruct((2 * NP, D), jnp.float32),
        mesh=plsc.VectorSubcoreMesh(**_SC_MESH),
        scratch_types=[
            pltpu.VMEM_SHARED((NP, D), jnp.float32),
            pltpu.VMEM((QC, CH), jnp.int32),
            pltpu.VMEM((CH, D), jnp.float32),
        ],
    )


def _sc_count(dsts):
    return _sc_count_kernel()(dsts)


@functools.cache
def _sc_scatter_kernel():
    return pl.kernel(
        _sc_scatter_body,
        out_type=jax.ShapeDtypeStruct((2 * NP, D), jnp.float32),
        mesh=plsc.VectorSubcoreMesh(**_SC_MESH),
        scratch_types=[
            pltpu.VMEM_SHARED((NP, D), jnp.float32),
            pltpu.VMEM((QC, CH), jnp.int32),
            pltpu.VMEM((QC, CH), jnp.int32),
            pltpu.VMEM((CH, D), jnp.float32),
            pltpu.VMEM((CH, D), jnp.float32),
            pltpu.SemaphoreType.DMA,
            pltpu.SemaphoreType.DMA,
        ],
    )


def _sc_scatter(hp, srcs, dsts):
    return _sc_scatter_kernel()(hp, srcs, dsts)


_RB = 400


def _dotT(a, b):
    return lax.dot_general(a, b, (((1,), (1,)), ((), ())),
                           preferred_element_type=jnp.float32)


def _mm_scale_body(x_ref, w_ref, cnt_ref, o_ref):
    dinv = lax.rsqrt(1.0 + cnt_ref[:, 0:1])
    o_ref[...] = _dotT(x_ref[...], w_ref[...]) * dinv


def _mm_scale(xs, w, cnt):
    grid = (2 * N) // _RB
    return pl.pallas_call(
        _mm_scale_body,
        grid=(grid,),
        in_specs=[
            pl.BlockSpec((_RB, D), lambda i: (i, 0)),
            pl.BlockSpec((D, D), lambda i: (0, 0)),
            pl.BlockSpec((_RB, D), lambda i: (i, 0)),
        ],
        out_specs=pl.BlockSpec((_RB, D), lambda i: (i, 0)),
        out_shape=jax.ShapeDtypeStruct((2 * N, D), jnp.float32),
    )(xs, w, cnt)


def _combine_mm_body(msg_ref, hp_ref, cnt_ref, b_ref, a_ref, w_ref, o_ref):
    dinv = lax.rsqrt(1.0 + cnt_ref[:, 0:1])
    t = dinv * (msg_ref[...] + hp_ref[...]) + b_ref[...]
    z = jnp.where(t >= 0.0, t, a_ref[...] * t)
    o_ref[...] = _dotT(z, w_ref[...]) * dinv


def _combine_mm(msg, hp, cnt, b, a, w):
    grid = (2 * N) // _RB
    return pl.pallas_call(
        _combine_mm_body,
        grid=(grid,),
        in_specs=[
            pl.BlockSpec((_RB, D), lambda i: (i, 0)),
            pl.BlockSpec((_RB, D), lambda i: (i, 0)),
            pl.BlockSpec((_RB, D), lambda i: (i, 0)),
            pl.BlockSpec((1, D), lambda i: (0, 0)),
            pl.BlockSpec((1, D), lambda i: (0, 0)),
            pl.BlockSpec((D, D), lambda i: (0, 0)),
        ],
        out_specs=pl.BlockSpec((_RB, D), lambda i: (i, 0)),
        out_shape=jax.ShapeDtypeStruct((2 * N, D), jnp.float32),
    )(msg, hp, cnt, b, a, w)


_GB = N // _RB


def _combine_segmax_body(msg_ref, hp_ref, cnt_ref, b_ref, a_ref, bb_ref,
                         z_ref, g_ref):
    i = pl.program_id(0)
    dinv = lax.rsqrt(1.0 + cnt_ref[:, 0:1])
    t = dinv * (msg_ref[...] + hp_ref[...]) + b_ref[...]
    z = jnp.where(t >= 0.0, t, a_ref[...] * t)
    z_ref[...] = z

    @pl.when(i % _GB == 0)
    def _():
        g_ref[...] = jnp.full((1, G, D), -jnp.inf, jnp.float32)

    bb = bb_ref[0]
    gmin = jnp.min(bb)
    gmax = jnp.max(bb)

    def _upd(g, acc):
        red = jnp.max(jnp.where(bb == g, z, -jnp.inf), axis=0, keepdims=True)
        sel = lax.broadcasted_iota(jnp.int32, (G, 1), 0) == g
        return jnp.where(sel, jnp.maximum(acc, red), acc)

    g_ref[0] = lax.fori_loop(gmin, gmax + 1, _upd, g_ref[0])


def _combine_segmax(msg, hp, cnt, b, a, batchc):
    return pl.pallas_call(
        _combine_segmax_body,
        grid=(2 * _GB,),
        in_specs=[
            pl.BlockSpec((_RB, D), lambda i: (i, 0)),
            pl.BlockSpec((_RB, D), lambda i: (i, 0)),
            pl.BlockSpec((_RB, D), lambda i: (i, 0)),
            pl.BlockSpec((1, D), lambda i: (0, 0)),
            pl.BlockSpec((1, D), lambda i: (0, 0)),
            pl.BlockSpec((1, _RB, 1), lambda i: (i % _GB, 0, 0)),
        ],
        out_specs=[
            pl.BlockSpec((_RB, D), lambda i: (i, 0)),
            pl.BlockSpec((1, G, D), lambda i: (i // _GB, 0, 0)),
        ],
        out_shape=[
            jax.ShapeDtypeStruct((2 * N, D), jnp.float32),
            jax.ShapeDtypeStruct((2, G, D), jnp.float32),
        ],
    )(msg, hp, cnt, b, a, batchc)


def _fc_body(x_ref, w1, b1, w2, b2, w3, b3, ws, bs, o_ref):
    x = x_ref[...]
    h = jnp.maximum(_dotT(x, w1[...]) + b1[...], 0.0)
    h = jnp.maximum(_dotT(h, w2[...]) + b2[...], 0.0)
    h = jnp.maximum(_dotT(h, w3[...]) + b3[...], 0.0)
    o_ref[...] = h + _dotT(x, ws[...]) + bs[...]


def _fc_block(x, p, rb):
    m = x.shape[0]
    args = [x]
    specs = [pl.BlockSpec((rb, D), lambda i: (i, 0))]
    for k in ("w1", "b1", "w2", "b2", "w3", "b3", "ws", "bs"):
        v = p[k]
        if v.ndim == 1:
            v = v.reshape(1, D)
            specs.append(pl.BlockSpec((1, D), lambda i: (0, 0)))
        else:
            specs.append(pl.BlockSpec((D, D), lambda i: (0, 0)))
        args.append(v)
    return pl.pallas_call(
        _fc_body,
        grid=(m // rb,),
        in_specs=specs,
        out_specs=pl.BlockSpec((rb, D), lambda i: (i, 0)),
        out_shape=jax.ShapeDtypeStruct((m, D), jnp.float32),
    )(*args)


def _infonce_body(an_ref, z_ref, b_ref, w1, b1, w2, b2, w3, b3, ws, bs,
                  o_ref, accA, accB, accC):
    j = pl.program_id(1)

    @pl.when(j == 0)
    def _():
        accA[...] = jnp.zeros_like(accA)
        accB[...] = jnp.zeros_like(accB)
        accC[...] = jnp.zeros_like(accC)

    an = an_ref[0]
    an = an / (jnp.sqrt(jnp.sum(an * an, axis=1, keepdims=True)) + 1e-12)
    x = z_ref[...]
    h = jnp.maximum(_dotT(x, w1[...]) + b1[...], 0.0)
    h = jnp.maximum(_dotT(h, w2[...]) + b2[...], 0.0)
    h = jnp.maximum(_dotT(h, w3[...]) + b3[...], 0.0)
    sm = h + _dotT(x, ws[...]) + bs[...]
    sm = sm / (jnp.sqrt(jnp.sum(sm * sm, axis=1, keepdims=True)) + 1e-12)
    sim = _dotT(an, sm) / TAU
    mask = b_ref[0] == lax.broadcasted_iota(jnp.int32, (G, 1), 0)
    accA[...] += jnp.sum(jnp.exp(sim), axis=1, keepdims=True)
    accB[...] += jnp.sum(jnp.where(mask, sim, 0.0), axis=1, keepdims=True)
    accC[...] += jnp.sum(mask.astype(jnp.float32), axis=1, keepdims=True)

    @pl.when(j == pl.num_programs(1) - 1)
    def _():
        per = accB[...] / accC[...] - jnp.log(accA[...])
        o_ref[...] = jnp.broadcast_to(-jnp.sum(per) / G, (1, 8, D))


def _infonce(anchors, zs, batch3i, prm):
    args = [anchors, zs, batch3i]
    specs = [
        pl.BlockSpec((1, G, D), lambda p, j: (p, 0, 0)),
        pl.BlockSpec((_RB, D), lambda p, j: ((1 - p) * _GB + j, 0)),
        pl.BlockSpec((1, 1, _RB), lambda p, j: (j, 0, 0)),
    ]
    for k in ("w1", "b1", "w2", "b2", "w3", "b3", "ws", "bs"):
        v = prm[k]
        if v.ndim == 1:
            v = v.reshape(1, D)
            specs.append(pl.BlockSpec((1, D), lambda p, j: (0, 0)))
        else:
            specs.append(pl.BlockSpec((D, D), lambda p, j: (0, 0)))
        args.append(v)
    return pl.pallas_call(
        _infonce_body,
        grid=(2, N // _RB),
        in_specs=specs,
        out_specs=pl.BlockSpec((1, 8, D), lambda p, j: (p, 0, 0)),
        out_shape=jax.ShapeDtypeStruct((2, 8, D), jnp.float32),
        scratch_shapes=[
            pltpu.VMEM((G, 1), jnp.float32),
            pltpu.VMEM((G, 1), jnp.float32),
            pltpu.VMEM((G, 1), jnp.float32),
        ],
    )(*args)


def _unpad(v):
    return jnp.concatenate([v[:N], v[NP:NP + N]])


def _chunked(idx, pad):
    body = idx.reshape(NS, EPT)
    tail = jnp.broadcast_to(pad, (NS, EPP - EPT))
    return jnp.concatenate([body, tail], axis=1).reshape(NS * NCH, CH)


def kernel(x, x1, params, edge, edge1, batch):
    dpad = N + jnp.arange(EPP - EPT, dtype=jnp.int32) % (NP - N)
    spad = jnp.arange(EPP - EPT, dtype=jnp.int32) % N
    srcs = jnp.concatenate(
        [_chunked(edge[0], spad), _chunked(edge1[0] + N, spad + N)])
    dsts = jnp.concatenate(
        [_chunked(edge[1], dpad), _chunked(edge1[1], dpad)])

    cnt = _unpad(_sc_count(dsts))

    xs = jnp.concatenate([x, x1], axis=0)
    a = params["prelu_a"].reshape(1, D)
    b0 = params["conv0_b"].reshape(1, D)
    b1 = params["conv1_b"].reshape(1, D)

    hp = _mm_scale(xs, params["conv0_w"], cnt)
    msg = _unpad(_sc_scatter(hp, srcs, dsts))
    hp = _combine_mm(msg, hp, cnt, b0, a, params["conv1_w"])
    msg = _unpad(_sc_scatter(hp, srcs, dsts))
    zs, gs = _combine_segmax(msg, hp, cnt, b1, a,
                             batch.reshape(_GB, _RB, 1))

    pro_gs = _fc_block(gs.reshape(2 * G, D), params["global"], 2 * G)

    batch3i = batch.reshape(N // _RB, 1, _RB)
    ls = _infonce(pro_gs.reshape(2, G, D), zs, batch3i, params["local"])
    loss = 0.5 * (ls[0, 0, 0] + ls[1, 0, 0])
    return (loss, zs[:N], gs[0])
